# baseline (speedup 1.0000x reference)
import functools

import jax
import jax.numpy as jnp
from jax import lax
from jax.experimental import pallas as pl
from jax.experimental.pallas import tpu as pltpu
from jax.experimental.pallas import tpu_sc as plsc

B, S, D = 1, 2048, 1024
E, TOPK = 8, 2
H = 4 * D
T = S * TOPK

RCHUNK = 128
BM = 128
NB = T // BM
NW = NB + E - 1
HC = 2048
NH = H // HC

NC, NS, LANES = 2, 16, 16
NWORK = NC * NS
CHUNK = T // NWORK
SUB = 64


def _router_body(x_ref, wr_ref, logits_ref, e_ref, r_ref, p_ref, cnt_ref,
                 run_ref):
    g = pl.program_id(0)

    @pl.when(g == 0)
    def _():
        run_ref[...] = jnp.zeros_like(run_ref)

    xg = x_ref[...]
    lg = jnp.dot(xg, wr_ref[...], preferred_element_type=jnp.float32)
    logits_ref[...] = lg

    idx = lax.broadcasted_iota(jnp.int32, (RCHUNK, E), 1)
    m0 = jnp.max(lg, axis=1, keepdims=True)
    e0 = jnp.min(jnp.where(lg == m0, idx, E), axis=1, keepdims=True)
    lg1 = jnp.where(idx == e0, -jnp.inf, lg)
    m1 = jnp.max(lg1, axis=1, keepdims=True)
    e1 = jnp.min(jnp.where(lg1 == m1, idx, E), axis=1, keepdims=True)

    d = jnp.exp(m1 - m0)
    p0 = 1.0 / (1.0 + d)
    p1 = d / (1.0 + d)
    p_ref[...] = jnp.concatenate([p0, p1], axis=1)
    e_ref[...] = jnp.concatenate([e0, e1], axis=1)

    onehot = (idx == e0).astype(jnp.float32) + (idx == e1).astype(jnp.float32)
    rr = lax.broadcasted_iota(jnp.int32, (RCHUNK, RCHUNK), 0)
    cc = lax.broadcasted_iota(jnp.int32, (RCHUNK, RCHUNK), 1)
    ltri = (cc < rr).astype(jnp.float32)
    pref = jnp.dot(ltri, onehot, preferred_element_type=jnp.float32)
    pref = pref + run_ref[...]

    r0 = jnp.sum(jnp.where(idx == e0, pref, 0.0), axis=1, keepdims=True)
    r1 = jnp.sum(jnp.where(idx == e1, pref, 0.0), axis=1, keepdims=True)
    r_ref[...] = jnp.concatenate([r0, r1], axis=1).astype(jnp.int32)

    run_ref[...] += jnp.sum(onehot, axis=0, keepdims=True)
    cnt_ref[...] = run_ref[...].astype(jnp.int32)


def _router(x2d, wr):
    grid = (S // RCHUNK,)
    return pl.pallas_call(
        _router_body,
        grid=grid,
        in_specs=[
            pl.BlockSpec((RCHUNK, D), lambda g: (g, 0)),
            pl.BlockSpec((D, E), lambda g: (0, 0)),
        ],
        out_specs=[
            pl.BlockSpec((RCHUNK, E), lambda g: (g, 0)),
            pl.BlockSpec((RCHUNK, TOPK), lambda g: (g, 0)),
            pl.BlockSpec((RCHUNK, TOPK), lambda g: (g, 0)),
            pl.BlockSpec((RCHUNK, TOPK), lambda g: (g, 0)),
            pl.BlockSpec((1, E), lambda g: (0, 0)),
        ],
        out_shape=[
            jax.ShapeDtypeStruct((S, E), jnp.float32),
            jax.ShapeDtypeStruct((S, TOPK), jnp.int32),
            jax.ShapeDtypeStruct((S, TOPK), jnp.int32),
            jax.ShapeDtypeStruct((S, TOPK), jnp.float32),
            jax.ShapeDtypeStruct((1, E), jnp.int32),
        ],
        scratch_shapes=[pltpu.VMEM((1, E), jnp.float32)],
        compiler_params=pltpu.CompilerParams(
            dimension_semantics=("arbitrary",)),
    )(x2d, wr)


def _build_meta(starts9):
    s = starts9
    bstart = jnp.arange(NB, dtype=jnp.int32) * BM
    g_first = jnp.sum(s[None, :E] <= bstart[:, None], axis=1) - 1
    g_last = jnp.sum(s[None, :E] <= (bstart + BM - 1)[:, None], axis=1) - 1
    nb_items = g_last - g_first + 1
    cum = jnp.concatenate([jnp.zeros(1, jnp.int32),
                           jnp.cumsum(nb_items).astype(jnp.int32)])
    ntot = cum[NB]
    w = jnp.arange(NW, dtype=jnp.int32)
    bw = jnp.sum(cum[None, :NB] <= w[:, None], axis=1) - 1
    bw = jnp.clip(bw, 0, NB - 1)
    gw = jnp.take(g_first, bw) + (w - jnp.take(cum, bw))
    valid = w < ntot
    g_dummy = g_last[NB - 1]
    blk = jnp.where(valid, bw, NB - 1)
    grp = jnp.where(valid, jnp.clip(gw, 0, E - 1), g_dummy)
    lo = jnp.where(valid, jnp.maximum(jnp.take(s, grp), blk * BM), 0)
    hi = jnp.where(valid, jnp.minimum(jnp.take(s, grp + 1), (blk + 1) * BM), 0)
    first = (valid & (w == jnp.take(cum, bw))).astype(jnp.int32)
    return jnp.stack([blk, grp, lo, hi, first]).astype(jnp.int32)


def _dest_body(st_ref, e_ref, r_ref, out_ref):
    e = e_ref[...]
    acc = r_ref[...]
    for g in range(E):
        acc = acc + jnp.where(e == g, st_ref[g], 0)
    out_ref[...] = acc


def _dest(starts8, eouts, ranks):
    grid_spec = pltpu.PrefetchScalarGridSpec(
        num_scalar_prefetch=1,
        grid=(1,),
        in_specs=[
            pl.BlockSpec((S, TOPK), lambda i, m: (0, 0)),
            pl.BlockSpec((S, TOPK), lambda i, m: (0, 0)),
        ],
        out_specs=pl.BlockSpec((S, TOPK), lambda i, m: (0, 0)),
    )
    return pl.pallas_call(
        _dest_body,
        grid_spec=grid_spec,
        out_shape=jax.ShapeDtypeStruct((S, TOPK), jnp.int32),
    )(starts8, eouts, ranks)


def _dispatch_sc(dest, src, x2d):
    mesh = plsc.VectorSubcoreMesh(core_axis_name="c", subcore_axis_name="s")

    @functools.partial(
        pl.kernel,
        mesh=mesh,
        out_type=jax.ShapeDtypeStruct((T, D), jnp.float32),
        scratch_types=[
            pltpu.VMEM((CHUNK // SUB, SUB), jnp.int32),
            pltpu.VMEM((CHUNK // SUB, SUB), jnp.int32),
            pltpu.VMEM((SUB, D), jnp.float32),
            pltpu.SemaphoreType.DMA,
        ],
    )
    def body(dest_hbm, src_hbm, x_hbm, xs_hbm, dest_v, src_v, rows_v, sem):
        wid = lax.axis_index("s") * NC + lax.axis_index("c")
        base = wid * CHUNK
        for sub in range(CHUNK // SUB):
            pltpu.sync_copy(dest_hbm.at[pl.ds(base + sub * SUB, SUB)],
                            dest_v.at[sub])
            pltpu.sync_copy(src_hbm.at[pl.ds(base + sub * SUB, SUB)],
                            src_v.at[sub])
            pltpu.async_copy(x_hbm.at[src_v.at[sub]], rows_v, sem).wait()
            pltpu.async_copy(rows_v, xs_hbm.at[dest_v.at[sub]], sem).wait()

    return body(dest, src, x2d)


def _ffn_body(meta_ref, xs_ref, w1_ref, b1_ref, w2_ref, b2_ref, out_ref,
              acc_ref):
    h = pl.program_id(0)
    w = pl.program_id(1)
    blk = meta_ref[0, w]
    lo = meta_ref[2, w]
    hi = meta_ref[3, w]
    first = meta_ref[4, w]
    off = blk * BM

    @pl.when(jnp.logical_and(h == 0, first == 1))
    def _():
        acc_ref[pl.ds(off, BM), :] = jnp.zeros((BM, D), jnp.float32)

    @pl.when(hi > lo)
    def _():
        rows = off + lax.broadcasted_iota(jnp.int32, (BM, 1), 0)
        mask = jnp.logical_and(rows >= lo, rows < hi).astype(jnp.float32)
        hh = jnp.dot(xs_ref[...], w1_ref[0],
                     preferred_element_type=jnp.float32) + b1_ref[0]
        hh = hh * 0.5 * (1.0 + lax.erf(hh * 0.7071067811865476))
        yp = jnp.dot(hh, w2_ref[0], preferred_element_type=jnp.float32)

        @pl.when(h == 0)
        def _():
            acc_ref[pl.ds(off, BM), :] += mask * b2_ref[0]

        acc_ref[pl.ds(off, BM), :] += mask * yp

    @pl.when(h == NH - 1)
    def _():
        out_ref[...] = acc_ref[pl.ds(off, BM), :]


def _ffn(meta, xs, w1, b1, w2, b2):
    grid_spec = pltpu.PrefetchScalarGridSpec(
        num_scalar_prefetch=1,
        grid=(NH, NW),
        in_specs=[
            pl.BlockSpec((BM, D), lambda h, w, m: (m[0, w], 0)),
            pl.BlockSpec((1, D, HC), lambda h, w, m: (m[1, w], 0, h)),
            pl.BlockSpec((1, 1, HC), lambda h, w, m: (m[1, w], 0, h)),
            pl.BlockSpec((1, HC, D), lambda h, w, m: (m[1, w], h, 0)),
            pl.BlockSpec((1, 1, D), lambda h, w, m: (m[1, w], 0, 0)),
        ],
        out_specs=pl.BlockSpec((BM, D), lambda h, w, m: (m[0, w], 0)),
        scratch_shapes=[pltpu.VMEM((T, D), jnp.float32)],
    )
    return pl.pallas_call(
        _ffn_body,
        grid_spec=grid_spec,
        out_shape=jax.ShapeDtypeStruct((T, D), jnp.float32),
        compiler_params=pltpu.CompilerParams(
            dimension_semantics=("arbitrary", "arbitrary")),
    )(meta, xs, w1, b1, w2, b2)


def _collect_sc(dest, ys):
    mesh = plsc.VectorSubcoreMesh(core_axis_name="c", subcore_axis_name="s")

    @functools.partial(
        pl.kernel,
        mesh=mesh,
        out_type=jax.ShapeDtypeStruct((T, D), jnp.float32),
        scratch_types=[
            pltpu.VMEM((CHUNK // SUB, SUB), jnp.int32),
            pltpu.VMEM((SUB, D), jnp.float32),
            pltpu.SemaphoreType.DMA,
        ],
    )
    def body(dest_hbm, ys_hbm, yo_hbm, dest_v, rows_v, sem):
        wid = lax.axis_index("s") * NC + lax.axis_index("c")
        base = wid * CHUNK
        for sub in range(CHUNK // SUB):
            pltpu.sync_copy(dest_hbm.at[pl.ds(base + sub * SUB, SUB)],
                            dest_v.at[sub])
            pltpu.async_copy(ys_hbm.at[dest_v.at[sub]], rows_v, sem).wait()
            pltpu.sync_copy(rows_v, yo_hbm.at[pl.ds(base + sub * SUB, SUB)])

    return body(dest, ys)


def _combine_body(yp_ref, p_ref, g_ref, b_ref, out_ref):
    y = yp_ref[...]
    p = p_ref[...]
    s = p[:, 0:1] * y[:, :D] + p[:, 1:2] * y[:, D:]
    mu = jnp.mean(s, axis=1, keepdims=True)
    c = s - mu
    var = jnp.mean(c * c, axis=1, keepdims=True)
    out_ref[...] = c * lax.rsqrt(var + 1e-5) * g_ref[...] + b_ref[...]


def _combine_ln(ypairs, probs, gamma, beta):
    LB = 256
    return pl.pallas_call(
        _combine_body,
        grid=(S // LB,),
        in_specs=[
            pl.BlockSpec((LB, 2 * D), lambda g: (g, 0)),
            pl.BlockSpec((LB, TOPK), lambda g: (g, 0)),
            pl.BlockSpec((1, D), lambda g: (0, 0)),
            pl.BlockSpec((1, D), lambda g: (0, 0)),
        ],
        out_specs=pl.BlockSpec((LB, D), lambda g: (g, 0)),
        out_shape=jax.ShapeDtypeStruct((S, D), jnp.float32),
        compiler_params=pltpu.CompilerParams(
            dimension_semantics=("parallel",)),
    )(ypairs, probs, gamma, beta)


def kernel(x, Wr, W1, b1, W2, b2, gamma, beta):
    x2d = x.reshape(S, D)
    logits, eouts, ranks, probs, counts = _router(x2d, Wr)

    counts8 = counts.reshape(E)
    starts9 = jnp.concatenate([jnp.zeros(1, jnp.int32),
                               jnp.cumsum(counts8).astype(jnp.int32)])
    meta = _build_meta(starts9)

    dest = _dest(starts9[:E], eouts, ranks).reshape(T)
    src = (jnp.arange(T, dtype=jnp.int32) // 2)
    xs = _dispatch_sc(dest, src, x2d)
    ys = _ffn(meta, xs, W1, b1.reshape(E, 1, H), W2, b2.reshape(E, 1, D))
    yo = _collect_sc(dest, ys)

    out = _combine_ln(yo.reshape(S, 2 * D), probs,
                      gamma.reshape(1, D), beta.reshape(1, D))
    return out.reshape(B, S, D), logits.reshape(B, S, E)


# speedup vs baseline: 1.1080x; 1.1080x over previous
import functools

import jax
import jax.numpy as jnp
from jax import lax
from jax.experimental import pallas as pl
from jax.experimental.pallas import tpu as pltpu
from jax.experimental.pallas import tpu_sc as plsc

B, S, D = 1, 2048, 1024
E, TOPK = 8, 2
H = 4 * D
T = S * TOPK

RCHUNK = 128
BM = 256
NB = T // BM
NW = NB + E - 1
HC = 2048
NH = H // HC

NC, NS, LANES = 2, 16, 16
NWORK = NC * NS
CHUNK = T // NWORK
SUB = 64


def _router_body(x_ref, wr_ref, logits_ref, e_ref, r_ref, p_ref, cnt_ref,
                 run_ref):
    g = pl.program_id(0)

    @pl.when(g == 0)
    def _():
        run_ref[...] = jnp.zeros_like(run_ref)

    xg = x_ref[...]
    lg = jnp.dot(xg, wr_ref[...], preferred_element_type=jnp.float32)
    logits_ref[...] = lg

    idx = lax.broadcasted_iota(jnp.int32, (RCHUNK, E), 1)
    m0 = jnp.max(lg, axis=1, keepdims=True)
    e0 = jnp.min(jnp.where(lg == m0, idx, E), axis=1, keepdims=True)
    lg1 = jnp.where(idx == e0, -jnp.inf, lg)
    m1 = jnp.max(lg1, axis=1, keepdims=True)
    e1 = jnp.min(jnp.where(lg1 == m1, idx, E), axis=1, keepdims=True)

    d = jnp.exp(m1 - m0)
    p0 = 1.0 / (1.0 + d)
    p1 = d / (1.0 + d)
    p_ref[...] = jnp.concatenate([p0, p1], axis=1)
    e_ref[...] = jnp.concatenate([e0, e1], axis=1)

    onehot = (idx == e0).astype(jnp.float32) + (idx == e1).astype(jnp.float32)
    rr = lax.broadcasted_iota(jnp.int32, (RCHUNK, RCHUNK), 0)
    cc = lax.broadcasted_iota(jnp.int32, (RCHUNK, RCHUNK), 1)
    ltri = (cc < rr).astype(jnp.float32)
    pref = jnp.dot(ltri, onehot, preferred_element_type=jnp.float32)
    pref = pref + run_ref[...]

    r0 = jnp.sum(jnp.where(idx == e0, pref, 0.0), axis=1, keepdims=True)
    r1 = jnp.sum(jnp.where(idx == e1, pref, 0.0), axis=1, keepdims=True)
    r_ref[...] = jnp.concatenate([r0, r1], axis=1).astype(jnp.int32)

    run_ref[...] += jnp.sum(onehot, axis=0, keepdims=True)
    cnt_ref[...] = run_ref[...].astype(jnp.int32)


def _router(x2d, wr):
    grid = (S // RCHUNK,)
    return pl.pallas_call(
        _router_body,
        grid=grid,
        in_specs=[
            pl.BlockSpec((RCHUNK, D), lambda g: (g, 0)),
            pl.BlockSpec((D, E), lambda g: (0, 0)),
        ],
        out_specs=[
            pl.BlockSpec((RCHUNK, E), lambda g: (g, 0)),
            pl.BlockSpec((RCHUNK, TOPK), lambda g: (g, 0)),
            pl.BlockSpec((RCHUNK, TOPK), lambda g: (g, 0)),
            pl.BlockSpec((RCHUNK, TOPK), lambda g: (g, 0)),
            pl.BlockSpec((1, E), lambda g: (0, 0)),
        ],
        out_shape=[
            jax.ShapeDtypeStruct((S, E), jnp.float32),
            jax.ShapeDtypeStruct((S, TOPK), jnp.int32),
            jax.ShapeDtypeStruct((S, TOPK), jnp.int32),
            jax.ShapeDtypeStruct((S, TOPK), jnp.float32),
            jax.ShapeDtypeStruct((1, E), jnp.int32),
        ],
        scratch_shapes=[pltpu.VMEM((1, E), jnp.float32)],
        compiler_params=pltpu.CompilerParams(
            dimension_semantics=("arbitrary",)),
    )(x2d, wr)


def _build_meta(starts9):
    s = starts9
    bstart = jnp.arange(NB, dtype=jnp.int32) * BM
    g_first = jnp.sum(s[None, :E] <= bstart[:, None], axis=1) - 1
    g_last = jnp.sum(s[None, :E] <= (bstart + BM - 1)[:, None], axis=1) - 1
    nb_items = g_last - g_first + 1
    cum = jnp.concatenate([jnp.zeros(1, jnp.int32),
                           jnp.cumsum(nb_items).astype(jnp.int32)])
    ntot = cum[NB]
    w = jnp.arange(NW, dtype=jnp.int32)
    bw = jnp.sum(cum[None, :NB] <= w[:, None], axis=1) - 1
    bw = jnp.clip(bw, 0, NB - 1)
    gw = jnp.take(g_first, bw) + (w - jnp.take(cum, bw))
    valid = w < ntot
    g_dummy = g_last[NB - 1]
    blk = jnp.where(valid, bw, NB - 1)
    grp = jnp.where(valid, jnp.clip(gw, 0, E - 1), g_dummy)
    lo = jnp.where(valid, jnp.maximum(jnp.take(s, grp), blk * BM), 0)
    hi = jnp.where(valid, jnp.minimum(jnp.take(s, grp + 1), (blk + 1) * BM), 0)
    first = (valid & (w == jnp.take(cum, bw))).astype(jnp.int32)
    return jnp.stack([blk, grp, lo, hi, first]).astype(jnp.int32)


def _dest_body(st_ref, e_ref, r_ref, out_ref):
    e = e_ref[...]
    acc = r_ref[...]
    for g in range(E):
        acc = acc + jnp.where(e == g, st_ref[g], 0)
    out_ref[...] = acc


def _dest(starts8, eouts, ranks):
    grid_spec = pltpu.PrefetchScalarGridSpec(
        num_scalar_prefetch=1,
        grid=(1,),
        in_specs=[
            pl.BlockSpec((S, TOPK), lambda i, m: (0, 0)),
            pl.BlockSpec((S, TOPK), lambda i, m: (0, 0)),
        ],
        out_specs=pl.BlockSpec((S, TOPK), lambda i, m: (0, 0)),
    )
    return pl.pallas_call(
        _dest_body,
        grid_spec=grid_spec,
        out_shape=jax.ShapeDtypeStruct((S, TOPK), jnp.int32),
    )(starts8, eouts, ranks)


def _dispatch_sc(dest, src, x2d):
    mesh = plsc.VectorSubcoreMesh(core_axis_name="c", subcore_axis_name="s")

    @functools.partial(
        pl.kernel,
        mesh=mesh,
        out_type=jax.ShapeDtypeStruct((T, D), jnp.float32),
        scratch_types=[
            pltpu.VMEM((CHUNK // SUB, SUB), jnp.int32),
            pltpu.VMEM((CHUNK // SUB, SUB), jnp.int32),
            pltpu.VMEM((SUB, D), jnp.float32),
            pltpu.SemaphoreType.DMA,
        ],
    )
    def body(dest_hbm, src_hbm, x_hbm, xs_hbm, dest_v, src_v, rows_v, sem):
        wid = lax.axis_index("s") * NC + lax.axis_index("c")
        base = wid * CHUNK
        for sub in range(CHUNK // SUB):
            pltpu.sync_copy(dest_hbm.at[pl.ds(base + sub * SUB, SUB)],
                            dest_v.at[sub])
            pltpu.sync_copy(src_hbm.at[pl.ds(base + sub * SUB, SUB)],
                            src_v.at[sub])
            pltpu.async_copy(x_hbm.at[src_v.at[sub]], rows_v, sem).wait()
            pltpu.async_copy(rows_v, xs_hbm.at[dest_v.at[sub]], sem).wait()

    return body(dest, src, x2d)


def _ffn_body(meta_ref, xs_ref, w1_ref, b1_ref, w2_ref, b2_ref, out_ref,
              acc_ref):
    h = pl.program_id(0)
    w = pl.program_id(1)
    blk = meta_ref[0, w]
    lo = meta_ref[2, w]
    hi = meta_ref[3, w]
    first = meta_ref[4, w]
    off = blk * BM

    @pl.when(jnp.logical_and(h == 0, first == 1))
    def _():
        acc_ref[pl.ds(off, BM), :] = jnp.zeros((BM, D), jnp.float32)

    @pl.when(hi > lo)
    def _():
        rows = off + lax.broadcasted_iota(jnp.int32, (BM, 1), 0)
        mask = jnp.logical_and(rows >= lo, rows < hi).astype(jnp.float32)
        hh = jnp.dot(xs_ref[...], w1_ref[0],
                     preferred_element_type=jnp.float32) + b1_ref[0]
        hh = hh * 0.5 * (1.0 + lax.erf(hh * 0.7071067811865476))
        yp = jnp.dot(hh, w2_ref[0], preferred_element_type=jnp.float32)

        @pl.when(h == 0)
        def _():
            acc_ref[pl.ds(off, BM), :] += mask * b2_ref[0]

        acc_ref[pl.ds(off, BM), :] += mask * yp

    @pl.when(h == NH - 1)
    def _():
        out_ref[...] = acc_ref[pl.ds(off, BM), :]


def _ffn(meta, xs, w1, b1, w2, b2):
    grid_spec = pltpu.PrefetchScalarGridSpec(
        num_scalar_prefetch=1,
        grid=(NH, NW),
        in_specs=[
            pl.BlockSpec((BM, D), lambda h, w, m: (m[0, w], 0)),
            pl.BlockSpec((1, D, HC), lambda h, w, m: (m[1, w], 0, h)),
            pl.BlockSpec((1, 1, HC), lambda h, w, m: (m[1, w], 0, h)),
            pl.BlockSpec((1, HC, D), lambda h, w, m: (m[1, w], h, 0)),
            pl.BlockSpec((1, 1, D), lambda h, w, m: (m[1, w], 0, 0)),
        ],
        out_specs=pl.BlockSpec((BM, D), lambda h, w, m: (m[0, w], 0)),
        scratch_shapes=[pltpu.VMEM((T, D), jnp.float32)],
    )
    return pl.pallas_call(
        _ffn_body,
        grid_spec=grid_spec,
        out_shape=jax.ShapeDtypeStruct((T, D), jnp.float32),
        compiler_params=pltpu.CompilerParams(
            dimension_semantics=("arbitrary", "arbitrary")),
    )(meta, xs, w1, b1, w2, b2)


def _collect_sc(dest, ys):
    mesh = plsc.VectorSubcoreMesh(core_axis_name="c", subcore_axis_name="s")

    @functools.partial(
        pl.kernel,
        mesh=mesh,
        out_type=jax.ShapeDtypeStruct((T, D), jnp.float32),
        scratch_types=[
            pltpu.VMEM((CHUNK // SUB, SUB), jnp.int32),
            pltpu.VMEM((SUB, D), jnp.float32),
            pltpu.SemaphoreType.DMA,
        ],
    )
    def body(dest_hbm, ys_hbm, yo_hbm, dest_v, rows_v, sem):
        wid = lax.axis_index("s") * NC + lax.axis_index("c")
        base = wid * CHUNK
        for sub in range(CHUNK // SUB):
            pltpu.sync_copy(dest_hbm.at[pl.ds(base + sub * SUB, SUB)],
                            dest_v.at[sub])
            pltpu.async_copy(ys_hbm.at[dest_v.at[sub]], rows_v, sem).wait()
            pltpu.sync_copy(rows_v, yo_hbm.at[pl.ds(base + sub * SUB, SUB)])

    return body(dest, ys)


def _combine_body(yp_ref, p_ref, g_ref, b_ref, out_ref):
    y = yp_ref[...].reshape(_LB, 2 * D)
    p = p_ref[...]
    s = p[:, 0:1] * y[:, :D] + p[:, 1:2] * y[:, D:]
    mu = jnp.mean(s, axis=1, keepdims=True)
    c = s - mu
    var = jnp.mean(c * c, axis=1, keepdims=True)
    out_ref[...] = c * lax.rsqrt(var + 1e-5) * g_ref[...] + b_ref[...]


_LB = 256


def _combine_ln(yo, probs, gamma, beta):
    return pl.pallas_call(
        _combine_body,
        grid=(S // _LB,),
        in_specs=[
            pl.BlockSpec((2 * _LB, D), lambda g: (g, 0)),
            pl.BlockSpec((_LB, TOPK), lambda g: (g, 0)),
            pl.BlockSpec((1, D), lambda g: (0, 0)),
            pl.BlockSpec((1, D), lambda g: (0, 0)),
        ],
        out_specs=pl.BlockSpec((_LB, D), lambda g: (g, 0)),
        out_shape=jax.ShapeDtypeStruct((S, D), jnp.float32),
        compiler_params=pltpu.CompilerParams(
            dimension_semantics=("parallel",)),
    )(yo, probs, gamma, beta)


def kernel(x, Wr, W1, b1, W2, b2, gamma, beta):
    x2d = x.reshape(S, D)
    logits, eouts, ranks, probs, counts = _router(x2d, Wr)

    counts8 = counts.reshape(E)
    starts9 = jnp.concatenate([jnp.zeros(1, jnp.int32),
                               jnp.cumsum(counts8).astype(jnp.int32)])
    meta = _build_meta(starts9)

    dest = _dest(starts9[:E], eouts, ranks).reshape(T)
    src = (jnp.arange(T, dtype=jnp.int32) // 2)
    xs = _dispatch_sc(dest, src, x2d)
    ys = _ffn(meta, xs, W1, b1.reshape(E, 1, H), W2, b2.reshape(E, 1, D))
    yo = _collect_sc(dest, ys)

    out = _combine_ln(yo, probs, gamma.reshape(1, D), beta.reshape(1, D))
    return out.reshape(B, S, D), logits.reshape(B, S, E)


# speedup vs baseline: 1.1370x; 1.0262x over previous
import functools

import jax
import jax.numpy as jnp
from jax import lax
from jax.experimental import pallas as pl
from jax.experimental.pallas import tpu as pltpu
from jax.experimental.pallas import tpu_sc as plsc

B, S, D = 1, 2048, 1024
E, TOPK = 8, 2
H = 4 * D
T = S * TOPK

RCHUNK = 256
BM = 256
NB = T // BM
NW = NB + E - 1
HC = 2048
NH = H // HC

NC, NS, LANES = 2, 16, 16
NWORK = NC * NS
CHUNK = T // NWORK
SUB = 64


def _router_body(x_ref, wr_ref, logits_ref, e_ref, r_ref, p_ref, cnt_ref,
                 run_ref):
    g = pl.program_id(0)

    @pl.when(g == 0)
    def _():
        run_ref[...] = jnp.zeros_like(run_ref)

    xg = x_ref[...]
    lg = jnp.dot(xg, wr_ref[...], preferred_element_type=jnp.float32)
    logits_ref[...] = lg

    idx = lax.broadcasted_iota(jnp.int32, (RCHUNK, E), 1)
    m0 = jnp.max(lg, axis=1, keepdims=True)
    e0 = jnp.min(jnp.where(lg == m0, idx, E), axis=1, keepdims=True)
    lg1 = jnp.where(idx == e0, -jnp.inf, lg)
    m1 = jnp.max(lg1, axis=1, keepdims=True)
    e1 = jnp.min(jnp.where(lg1 == m1, idx, E), axis=1, keepdims=True)

    d = jnp.exp(m1 - m0)
    p0 = 1.0 / (1.0 + d)
    p1 = d / (1.0 + d)
    p_ref[...] = jnp.concatenate([p0, p1], axis=1)
    e_ref[...] = jnp.concatenate([e0, e1], axis=1)

    onehot = (idx == e0).astype(jnp.float32) + (idx == e1).astype(jnp.float32)
    rr = lax.broadcasted_iota(jnp.int32, (RCHUNK, RCHUNK), 0)
    cc = lax.broadcasted_iota(jnp.int32, (RCHUNK, RCHUNK), 1)
    ltri = (cc < rr).astype(jnp.float32)
    pref = jnp.dot(ltri, onehot, preferred_element_type=jnp.float32)
    pref = pref + run_ref[...]

    r0 = jnp.sum(jnp.where(idx == e0, pref, 0.0), axis=1, keepdims=True)
    r1 = jnp.sum(jnp.where(idx == e1, pref, 0.0), axis=1, keepdims=True)
    r_ref[...] = jnp.concatenate([r0, r1], axis=1).astype(jnp.int32)

    run_ref[...] += jnp.sum(onehot, axis=0, keepdims=True)
    cnt_ref[...] = run_ref[...].astype(jnp.int32)


def _router(x2d, wr):
    grid = (S // RCHUNK,)
    return pl.pallas_call(
        _router_body,
        grid=grid,
        in_specs=[
            pl.BlockSpec((RCHUNK, D), lambda g: (g, 0)),
            pl.BlockSpec((D, E), lambda g: (0, 0)),
        ],
        out_specs=[
            pl.BlockSpec((RCHUNK, E), lambda g: (g, 0)),
            pl.BlockSpec((RCHUNK, TOPK), lambda g: (g, 0)),
            pl.BlockSpec((RCHUNK, TOPK), lambda g: (g, 0)),
            pl.BlockSpec((RCHUNK, TOPK), lambda g: (g, 0)),
            pl.BlockSpec((1, E), lambda g: (0, 0)),
        ],
        out_shape=[
            jax.ShapeDtypeStruct((S, E), jnp.float32),
            jax.ShapeDtypeStruct((S, TOPK), jnp.int32),
            jax.ShapeDtypeStruct((S, TOPK), jnp.int32),
            jax.ShapeDtypeStruct((S, TOPK), jnp.float32),
            jax.ShapeDtypeStruct((1, E), jnp.int32),
        ],
        scratch_shapes=[pltpu.VMEM((1, E), jnp.float32)],
        compiler_params=pltpu.CompilerParams(
            dimension_semantics=("arbitrary",)),
    )(x2d, wr)


def _build_meta(starts9):
    s = starts9
    bstart = jnp.arange(NB, dtype=jnp.int32) * BM
    g_first = jnp.sum(s[None, :E] <= bstart[:, None], axis=1) - 1
    g_last = jnp.sum(s[None, :E] <= (bstart + BM - 1)[:, None], axis=1) - 1
    nb_items = g_last - g_first + 1
    cum = jnp.concatenate([jnp.zeros(1, jnp.int32),
                           jnp.cumsum(nb_items).astype(jnp.int32)])
    ntot = cum[NB]
    w = jnp.arange(NW, dtype=jnp.int32)
    bw = jnp.sum(cum[None, :NB] <= w[:, None], axis=1) - 1
    bw = jnp.clip(bw, 0, NB - 1)
    gw = jnp.take(g_first, bw) + (w - jnp.take(cum, bw))
    valid = w < ntot
    g_dummy = g_last[NB - 1]
    blk = jnp.where(valid, bw, NB - 1)
    grp = jnp.where(valid, jnp.clip(gw, 0, E - 1), g_dummy)
    lo = jnp.where(valid, jnp.maximum(jnp.take(s, grp), blk * BM), 0)
    hi = jnp.where(valid, jnp.minimum(jnp.take(s, grp + 1), (blk + 1) * BM), 0)
    first = (valid & (w == jnp.take(cum, bw))).astype(jnp.int32)
    return jnp.stack([blk, grp, lo, hi, first]).astype(jnp.int32)


def _dest_body(st_ref, e_ref, r_ref, out_ref):
    e = e_ref[...]
    acc = r_ref[...]
    for g in range(E):
        acc = acc + jnp.where(e == g, st_ref[g], 0)
    out_ref[...] = acc


def _dest(starts8, eouts, ranks):
    grid_spec = pltpu.PrefetchScalarGridSpec(
        num_scalar_prefetch=1,
        grid=(1,),
        in_specs=[
            pl.BlockSpec((S, TOPK), lambda i, m: (0, 0)),
            pl.BlockSpec((S, TOPK), lambda i, m: (0, 0)),
        ],
        out_specs=pl.BlockSpec((S, TOPK), lambda i, m: (0, 0)),
    )
    return pl.pallas_call(
        _dest_body,
        grid_spec=grid_spec,
        out_shape=jax.ShapeDtypeStruct((S, TOPK), jnp.int32),
    )(starts8, eouts, ranks)


def _dispatch_sc(dest, src, x2d):
    mesh = plsc.VectorSubcoreMesh(core_axis_name="c", subcore_axis_name="s")

    @functools.partial(
        pl.kernel,
        mesh=mesh,
        out_type=jax.ShapeDtypeStruct((T, D), jnp.float32),
        scratch_types=[
            pltpu.VMEM((CHUNK // SUB, SUB), jnp.int32),
            pltpu.VMEM((CHUNK // SUB, SUB), jnp.int32),
            pltpu.VMEM((SUB, D), jnp.float32),
            pltpu.SemaphoreType.DMA,
        ],
    )
    def body(dest_hbm, src_hbm, x_hbm, xs_hbm, dest_v, src_v, rows_v, sem):
        wid = lax.axis_index("s") * NC + lax.axis_index("c")
        base = wid * CHUNK
        for sub in range(CHUNK // SUB):
            pltpu.sync_copy(dest_hbm.at[pl.ds(base + sub * SUB, SUB)],
                            dest_v.at[sub])
            pltpu.sync_copy(src_hbm.at[pl.ds(base + sub * SUB, SUB)],
                            src_v.at[sub])
            pltpu.async_copy(x_hbm.at[src_v.at[sub]], rows_v, sem).wait()
            pltpu.async_copy(rows_v, xs_hbm.at[dest_v.at[sub]], sem).wait()

    return body(dest, src, x2d)


def _ffn_body(meta_ref, xs_ref, w1_ref, b1_ref, w2_ref, b2_ref, out_ref,
              acc_ref):
    h = pl.program_id(0)
    w = pl.program_id(1)
    blk = meta_ref[0, w]
    lo = meta_ref[2, w]
    hi = meta_ref[3, w]
    first = meta_ref[4, w]
    off = blk * BM

    @pl.when(jnp.logical_and(h == 0, first == 1))
    def _():
        acc_ref[pl.ds(off, BM), :] = jnp.zeros((BM, D), jnp.float32)

    @pl.when(hi > lo)
    def _():
        rows = off + lax.broadcasted_iota(jnp.int32, (BM, 1), 0)
        mask = jnp.logical_and(rows >= lo, rows < hi).astype(jnp.float32)
        hh = jnp.dot(xs_ref[...], w1_ref[0],
                     preferred_element_type=jnp.float32) + b1_ref[0]
        hh = hh * 0.5 * (1.0 + lax.erf(hh * 0.7071067811865476))
        yp = jnp.dot(hh, w2_ref[0], preferred_element_type=jnp.float32)

        @pl.when(h == 0)
        def _():
            acc_ref[pl.ds(off, BM), :] += mask * b2_ref[0]

        acc_ref[pl.ds(off, BM), :] += mask * yp

    @pl.when(h == NH - 1)
    def _():
        out_ref[...] = acc_ref[pl.ds(off, BM), :]


def _ffn(meta, xs, w1, b1, w2, b2):
    grid_spec = pltpu.PrefetchScalarGridSpec(
        num_scalar_prefetch=1,
        grid=(NH, NW),
        in_specs=[
            pl.BlockSpec((BM, D), lambda h, w, m: (m[0, w], 0)),
            pl.BlockSpec((1, D, HC), lambda h, w, m: (m[1, w], 0, h)),
            pl.BlockSpec((1, 1, HC), lambda h, w, m: (m[1, w], 0, h)),
            pl.BlockSpec((1, HC, D), lambda h, w, m: (m[1, w], h, 0)),
            pl.BlockSpec((1, 1, D), lambda h, w, m: (m[1, w], 0, 0)),
        ],
        out_specs=pl.BlockSpec((BM, D), lambda h, w, m: (m[0, w], 0)),
        scratch_shapes=[pltpu.VMEM((T, D), jnp.float32)],
    )
    return pl.pallas_call(
        _ffn_body,
        grid_spec=grid_spec,
        out_shape=jax.ShapeDtypeStruct((T, D), jnp.float32),
        compiler_params=pltpu.CompilerParams(
            dimension_semantics=("arbitrary", "arbitrary")),
    )(meta, xs, w1, b1, w2, b2)


def _collect_sc(dest, ys):
    mesh = plsc.VectorSubcoreMesh(core_axis_name="c", subcore_axis_name="s")

    @functools.partial(
        pl.kernel,
        mesh=mesh,
        out_type=jax.ShapeDtypeStruct((T, D), jnp.float32),
        scratch_types=[
            pltpu.VMEM((CHUNK // SUB, SUB), jnp.int32),
            pltpu.VMEM((SUB, D), jnp.float32),
            pltpu.SemaphoreType.DMA,
        ],
    )
    def body(dest_hbm, ys_hbm, yo_hbm, dest_v, rows_v, sem):
        wid = lax.axis_index("s") * NC + lax.axis_index("c")
        base = wid * CHUNK
        for sub in range(CHUNK // SUB):
            pltpu.sync_copy(dest_hbm.at[pl.ds(base + sub * SUB, SUB)],
                            dest_v.at[sub])
            pltpu.async_copy(ys_hbm.at[dest_v.at[sub]], rows_v, sem).wait()
            pltpu.sync_copy(rows_v, yo_hbm.at[pl.ds(base + sub * SUB, SUB)])

    return body(dest, ys)


def _combine_body(yp_ref, p_ref, g_ref, b_ref, out_ref):
    y = yp_ref[...].reshape(_LB, 2 * D)
    p = p_ref[...]
    s = p[:, 0:1] * y[:, :D] + p[:, 1:2] * y[:, D:]
    mu = jnp.mean(s, axis=1, keepdims=True)
    c = s - mu
    var = jnp.mean(c * c, axis=1, keepdims=True)
    out_ref[...] = c * lax.rsqrt(var + 1e-5) * g_ref[...] + b_ref[...]


_LB = 256


def _combine_ln(yo, probs, gamma, beta):
    return pl.pallas_call(
        _combine_body,
        grid=(S // _LB,),
        in_specs=[
            pl.BlockSpec((2 * _LB, D), lambda g: (g, 0)),
            pl.BlockSpec((_LB, TOPK), lambda g: (g, 0)),
            pl.BlockSpec((1, D), lambda g: (0, 0)),
            pl.BlockSpec((1, D), lambda g: (0, 0)),
        ],
        out_specs=pl.BlockSpec((_LB, D), lambda g: (g, 0)),
        out_shape=jax.ShapeDtypeStruct((S, D), jnp.float32),
        compiler_params=pltpu.CompilerParams(
            dimension_semantics=("parallel",)),
    )(yo, probs, gamma, beta)


def kernel(x, Wr, W1, b1, W2, b2, gamma, beta):
    x2d = x.reshape(S, D)
    logits, eouts, ranks, probs, counts = _router(x2d, Wr)

    counts8 = counts.reshape(E)
    starts9 = jnp.concatenate([jnp.zeros(1, jnp.int32),
                               jnp.cumsum(counts8).astype(jnp.int32)])
    meta = _build_meta(starts9)

    dest = _dest(starts9[:E], eouts, ranks).reshape(T)
    src = (jnp.arange(T, dtype=jnp.int32) // 2)
    xs = _dispatch_sc(dest, src, x2d)
    ys = _ffn(meta, xs, W1, b1.reshape(E, 1, H), W2, b2.reshape(E, 1, D))
    yo = _collect_sc(dest, ys)

    out = _combine_ln(yo, probs, gamma.reshape(1, D), beta.reshape(1, D))
    return out.reshape(B, S, D), logits.reshape(B, S, E)
